# baseline (speedup 1.0000x reference)
import jax
import jax.numpy as jnp
from jax import lax
from jax.experimental import pallas as pl
from jax.experimental.pallas import tpu as pltpu
from jax.experimental.pallas import tpu_sc as plsc

N = 10000
E = 320000
D = 128
HD = D // 2
NC = 2
NS = 16
CHUNK = 128
K = 160
SCN = 8
NSUP = K // SCN
E_PAD = NS * K * CHUNK
ACC_N = 10240
ROWS_PER_TILE = ACC_N // NS
STG = N // NS
CW = 16
CZ = 64
IDX_BITS = 14


def _agg_body(tabs, pk, out0, out1, cnt0,
              pkc, dstc, rows_v, ones_v, czbuf, tab_sh, acc_sh, cnt_sh, sem,
              sem_i, sem_s):
  cid = lax.axis_index("c")
  sid = lax.axis_index("s")
  rbase = sid * ROWS_PER_TILE

  def fill_rows(i, carry):
    for j in range(HD // 16):
      rows_v[0, i, pl.ds(16 * j, 16)] = jnp.zeros((16,), jnp.float32)
    ones_v[i, :] = jnp.ones((CW,), jnp.float32)
    return carry
  lax.fori_loop(0, CHUNK, fill_rows, 0)

  @pl.when(cid == 0)
  def _():
    def fill_cz(i, carry):
      czbuf[i, :] = jnp.zeros((CW,), jnp.float32)
      return carry
    lax.fori_loop(0, CZ, fill_cz, 0)
    for m in range(ROWS_PER_TILE // CZ):
      pltpu.sync_copy(czbuf, cnt_sh.at[pl.ds(rbase + m * CZ, CZ)])

  pltpu.sync_copy(tabs.at[cid, pl.ds(sid * STG, STG)],
                  tab_sh.at[pl.ds(sid * STG, STG)])

  for m in range(ROWS_PER_TILE // CHUNK):
    pltpu.sync_copy(rows_v.at[0], acc_sh.at[pl.ds(rbase + m * CHUNK, CHUNK)])

  def unpack(q):
    for jj in range(SCN):
      for i in range(CHUNK // 16):
        v = pkc[q, jj, pl.ds(16 * i, 16)]
        dstc[q, jj, pl.ds(16 * i, 16)] = jnp.right_shift(v, IDX_BITS)
        pkc[q, jj, pl.ds(16 * i, 16)] = v & ((1 << IDX_BITS) - 1)

  def start_idx(q, s):
    pltpu.async_copy(pk.at[sid, pl.ds(s * SCN, SCN)], pkc.at[q], sem_i)

  def wait_idx(q, s):
    pltpu.make_async_copy(pk.at[sid, pl.ds(s * SCN, SCN)], pkc.at[q],
                          sem_i).wait()

  start_idx(0, 0)
  wait_idx(0, 0)
  unpack(0)
  plsc.subcore_barrier()

  pltpu.async_copy(tab_sh.at[pkc.at[0, 0]], rows_v.at[0], sem)

  def superstep(s, carry):
    p = s % 2

    @pl.when(s < NSUP - 1)
    def _():
      start_idx(1 - p, s + 1)

    for jj in range(SCN):
      b = jj % 2
      pltpu.make_async_copy(tab_sh.at[pkc.at[p, jj]], rows_v.at[b],
                            sem).wait()
      if jj > 0:
        pltpu.make_async_copy(rows_v.at[1 - b],
                              acc_sh.at[dstc.at[p, jj - 1]], sem_s).wait()
      else:
        @pl.when(s > 0)
        def _():
          pltpu.make_async_copy(rows_v.at[1 - b],
                                acc_sh.at[dstc.at[1 - p, SCN - 1]],
                                sem_s).wait()
      pltpu.async_copy(rows_v.at[b], acc_sh.at[dstc.at[p, jj]], sem_s,
                       add=True)

      @pl.when(cid == 0)
      def _():
        pltpu.sync_copy(ones_v, cnt_sh.at[dstc.at[p, jj]], add=True)

      if jj < SCN - 1:
        pltpu.async_copy(tab_sh.at[pkc.at[p, jj + 1]], rows_v.at[1 - b], sem)
      else:
        @pl.when(s < NSUP - 1)
        def _():
          pltpu.async_copy(tab_sh.at[pkc.at[1 - p, 0]], rows_v.at[1 - b],
                           sem)

      if jj == SCN - 2:
        @pl.when(s < NSUP - 1)
        def _():
          wait_idx(1 - p, s + 1)
          unpack(1 - p)

    return carry

  lax.fori_loop(0, NSUP, superstep, 0)
  pltpu.make_async_copy(rows_v.at[1], acc_sh.at[dstc.at[(NSUP - 1) % 2,
                                                        SCN - 1]],
                        sem_s).wait()
  plsc.subcore_barrier()

  def write_out(dst_hbm, src_sh):
    @pl.when(sid < NS - 1)
    def _():
      pltpu.sync_copy(src_sh.at[pl.ds(rbase, ROWS_PER_TILE)],
                      dst_hbm.at[pl.ds(rbase, ROWS_PER_TILE)])

    @pl.when(sid == NS - 1)
    def _():
      last = N - (NS - 1) * ROWS_PER_TILE
      pltpu.sync_copy(src_sh.at[pl.ds((NS - 1) * ROWS_PER_TILE, last)],
                      dst_hbm.at[pl.ds((NS - 1) * ROWS_PER_TILE, last)])

  @pl.when(cid == 0)
  def _():
    write_out(out0, acc_sh)
    write_out(cnt0, cnt_sh)

  @pl.when(cid == 1)
  def _():
    write_out(out1, acc_sh)


_agg = pl.kernel(
    _agg_body,
    out_type=(
        jax.ShapeDtypeStruct((N, HD), jnp.float32),
        jax.ShapeDtypeStruct((N, HD), jnp.float32),
        jax.ShapeDtypeStruct((N, CW), jnp.float32),
    ),
    mesh=plsc.VectorSubcoreMesh(core_axis_name="c", subcore_axis_name="s"),
    scratch_types=[
        pltpu.VMEM((2, SCN, CHUNK), jnp.int32),
        pltpu.VMEM((2, SCN, CHUNK), jnp.int32),
        pltpu.VMEM((2, CHUNK, HD), jnp.float32),
        pltpu.VMEM((CHUNK, CW), jnp.float32),
        pltpu.VMEM((CZ, CW), jnp.float32),
        pltpu.VMEM_SHARED((N, HD), jnp.float32),
        pltpu.VMEM_SHARED((ACC_N, HD), jnp.float32),
        pltpu.VMEM_SHARED((ACC_N, CW), jnp.float32),
        pltpu.SemaphoreType.DMA,
        pltpu.SemaphoreType.DMA,
        pltpu.SemaphoreType.DMA,
    ],
    compiler_params=pltpu.CompilerParams(use_tc_tiling_on_sc=False),
)


def _tc_layer(pa, pb, cnt, x2, w_l, w_r, b_l, fl):
  nb = 10
  br = N // nb

  def body(pa_ref, pb_ref, c_ref, x2_ref, wl_ref, wr_ref, b_ref, f_ref,
           o_ref):
    c = jnp.maximum(c_ref[...], 1.0)
    mean = jnp.concatenate([pa_ref[...] / c, pb_ref[...] / c], axis=1)
    xin = jnp.concatenate([x2_ref[0], x2_ref[1]], axis=1)
    dn = (((1,), (1,)), ((), ()))
    r = (lax.dot_general(mean, wl_ref[...], dn,
                         preferred_element_type=jnp.float32)
         + lax.dot_general(xin, wr_ref[...], dn,
                           preferred_element_type=jnp.float32)
         + b_ref[...])
    r = jnp.where(f_ref[...] > 0.5, jnp.maximum(r, 0.0), r)
    o_ref[0] = r[:, :HD]
    o_ref[1] = r[:, HD:]

  half_spec = pl.BlockSpec((br, HD), lambda i: (i, 0))
  split_spec = pl.BlockSpec((2, br, HD), lambda i: (0, i, 0))
  return pl.pallas_call(
      body,
      grid=(nb,),
      in_specs=[
          half_spec, half_spec,
          pl.BlockSpec((br, 1), lambda i: (i, 0)),
          split_spec,
          pl.BlockSpec((D, D), lambda i: (0, 0)),
          pl.BlockSpec((D, D), lambda i: (0, 0)),
          pl.BlockSpec((1, D), lambda i: (0, 0)),
          pl.BlockSpec((1, 1), lambda i: (0, 0)),
      ],
      out_specs=split_spec,
      out_shape=jax.ShapeDtypeStruct((2, N, HD), jnp.float32),
  )(pa, pb, cnt, x2, w_l, w_r, b_l.reshape(1, D), fl)


def kernel(x, edge_index, W1_l, b1_l, W1_r, W2_l, b2_l, W2_r):
  src = edge_index[0].astype(jnp.int32)
  dst = edge_index[1].astype(jnp.int32)
  packed = src | (dst << IDX_BITS)
  pk = jnp.concatenate(
      [packed, jnp.full((E_PAD - E,), N << IDX_BITS, jnp.int32)]
  ).reshape(NS, K, CHUNK)

  x2 = jnp.stack([x[:, :HD], x[:, HD:]])
  wls = jnp.stack([W1_l, W2_l])
  wrs = jnp.stack([W1_r, W2_r])
  bs = jnp.stack([b1_l, b2_l])
  fls = jnp.array([[[1.0]], [[0.0]]], jnp.float32)

  def step(h2, ws):
    w_l, w_r, b_l, fl = ws
    pa, pb, cnt = _agg(h2, pk)
    h2n = _tc_layer(pa, pb, cnt[:, :1], h2, w_l, w_r, b_l, fl)
    return h2n, 0

  out2, _ = lax.scan(step, x2, (wls, wrs, bs, fls))
  return jnp.concatenate([out2[0], out2[1]], axis=1)


# speedup vs baseline: 1.0022x; 1.0022x over previous
import jax
import jax.numpy as jnp
from jax import lax
from jax.experimental import pallas as pl
from jax.experimental.pallas import tpu as pltpu
from jax.experimental.pallas import tpu_sc as plsc

N = 10000
E = 320000
D = 128
HD = D // 2
NC = 2
NS = 16
CHUNK = 128
K = 160
SCN = 4
NSUP = K // SCN
E_PAD = NS * K * CHUNK
ACC_N = 10240
ROWS_PER_TILE = ACC_N // NS
STG = N // NS
CW = 16
CZ = 64
IDX_BITS = 14


def _agg_body(tabs, pk, out0, out1, cnt0,
              pkc, dstc, rows_v, ones_v, czbuf, tab_sh, acc_sh, cnt_sh, sem,
              sem_i, sem_s):
  cid = lax.axis_index("c")
  sid = lax.axis_index("s")
  rbase = sid * ROWS_PER_TILE

  def fill_rows(i, carry):
    for j in range(HD // 16):
      rows_v[0, i, pl.ds(16 * j, 16)] = jnp.zeros((16,), jnp.float32)
    ones_v[i, :] = jnp.ones((CW,), jnp.float32)
    return carry
  lax.fori_loop(0, CHUNK, fill_rows, 0)

  @pl.when(cid == 0)
  def _():
    def fill_cz(i, carry):
      czbuf[i, :] = jnp.zeros((CW,), jnp.float32)
      return carry
    lax.fori_loop(0, CZ, fill_cz, 0)
    for m in range(ROWS_PER_TILE // CZ):
      pltpu.sync_copy(czbuf, cnt_sh.at[pl.ds(rbase + m * CZ, CZ)])

  pltpu.sync_copy(tabs.at[cid, pl.ds(sid * STG, STG)],
                  tab_sh.at[pl.ds(sid * STG, STG)])

  for m in range(ROWS_PER_TILE // CHUNK):
    pltpu.sync_copy(rows_v.at[0], acc_sh.at[pl.ds(rbase + m * CHUNK, CHUNK)])

  def unpack(q):
    for jj in range(SCN):
      for i in range(CHUNK // 16):
        v = pkc[q, jj, pl.ds(16 * i, 16)]
        dstc[q, jj, pl.ds(16 * i, 16)] = jnp.right_shift(v, IDX_BITS)
        pkc[q, jj, pl.ds(16 * i, 16)] = v & ((1 << IDX_BITS) - 1)

  def start_idx(q, s):
    pltpu.async_copy(pk.at[sid, pl.ds(s * SCN, SCN)], pkc.at[q], sem_i)

  def wait_idx(q, s):
    pltpu.make_async_copy(pk.at[sid, pl.ds(s * SCN, SCN)], pkc.at[q],
                          sem_i).wait()

  start_idx(0, 0)
  wait_idx(0, 0)
  unpack(0)
  plsc.subcore_barrier()

  pltpu.async_copy(tab_sh.at[pkc.at[0, 0]], rows_v.at[0], sem)

  def superstep(s, carry):
    p = s % 2

    @pl.when(s < NSUP - 1)
    def _():
      start_idx(1 - p, s + 1)

    for jj in range(SCN):
      b = jj % 2
      pltpu.make_async_copy(tab_sh.at[pkc.at[p, jj]], rows_v.at[b],
                            sem).wait()
      if jj > 0:
        pltpu.make_async_copy(rows_v.at[1 - b],
                              acc_sh.at[dstc.at[p, jj - 1]], sem_s).wait()
      else:
        @pl.when(s > 0)
        def _():
          pltpu.make_async_copy(rows_v.at[1 - b],
                                acc_sh.at[dstc.at[1 - p, SCN - 1]],
                                sem_s).wait()
      pltpu.async_copy(rows_v.at[b], acc_sh.at[dstc.at[p, jj]], sem_s,
                       add=True)

      @pl.when(cid == 0)
      def _():
        pltpu.sync_copy(ones_v, cnt_sh.at[dstc.at[p, jj]], add=True)

      if jj < SCN - 1:
        pltpu.async_copy(tab_sh.at[pkc.at[p, jj + 1]], rows_v.at[1 - b], sem)
      else:
        @pl.when(s < NSUP - 1)
        def _():
          pltpu.async_copy(tab_sh.at[pkc.at[1 - p, 0]], rows_v.at[1 - b],
                           sem)

      if jj == SCN - 2:
        @pl.when(s < NSUP - 1)
        def _():
          wait_idx(1 - p, s + 1)
          unpack(1 - p)

    return carry

  lax.fori_loop(0, NSUP, superstep, 0)
  pltpu.make_async_copy(rows_v.at[1], acc_sh.at[dstc.at[(NSUP - 1) % 2,
                                                        SCN - 1]],
                        sem_s).wait()
  plsc.subcore_barrier()

  def write_out(dst_hbm, src_sh):
    @pl.when(sid < NS - 1)
    def _():
      pltpu.sync_copy(src_sh.at[pl.ds(rbase, ROWS_PER_TILE)],
                      dst_hbm.at[pl.ds(rbase, ROWS_PER_TILE)])

    @pl.when(sid == NS - 1)
    def _():
      last = N - (NS - 1) * ROWS_PER_TILE
      pltpu.sync_copy(src_sh.at[pl.ds((NS - 1) * ROWS_PER_TILE, last)],
                      dst_hbm.at[pl.ds((NS - 1) * ROWS_PER_TILE, last)])

  @pl.when(cid == 0)
  def _():
    write_out(out0, acc_sh)
    write_out(cnt0, cnt_sh)

  @pl.when(cid == 1)
  def _():
    write_out(out1, acc_sh)


_agg = pl.kernel(
    _agg_body,
    out_type=(
        jax.ShapeDtypeStruct((N, HD), jnp.float32),
        jax.ShapeDtypeStruct((N, HD), jnp.float32),
        jax.ShapeDtypeStruct((N, CW), jnp.float32),
    ),
    mesh=plsc.VectorSubcoreMesh(core_axis_name="c", subcore_axis_name="s"),
    scratch_types=[
        pltpu.VMEM((2, SCN, CHUNK), jnp.int32),
        pltpu.VMEM((2, SCN, CHUNK), jnp.int32),
        pltpu.VMEM((2, CHUNK, HD), jnp.float32),
        pltpu.VMEM((CHUNK, CW), jnp.float32),
        pltpu.VMEM((CZ, CW), jnp.float32),
        pltpu.VMEM_SHARED((N, HD), jnp.float32),
        pltpu.VMEM_SHARED((ACC_N, HD), jnp.float32),
        pltpu.VMEM_SHARED((ACC_N, CW), jnp.float32),
        pltpu.SemaphoreType.DMA,
        pltpu.SemaphoreType.DMA,
        pltpu.SemaphoreType.DMA,
    ],
    compiler_params=pltpu.CompilerParams(use_tc_tiling_on_sc=False),
)


def _tc_layer(pa, pb, cnt, x2, w_l, w_r, b_l, fl):
  nb = 10
  br = N // nb

  def body(pa_ref, pb_ref, c_ref, x2_ref, wl_ref, wr_ref, b_ref, f_ref,
           o_ref):
    c = jnp.maximum(c_ref[...], 1.0)
    mean = jnp.concatenate([pa_ref[...] / c, pb_ref[...] / c], axis=1)
    xin = jnp.concatenate([x2_ref[0], x2_ref[1]], axis=1)
    dn = (((1,), (1,)), ((), ()))
    r = (lax.dot_general(mean, wl_ref[...], dn,
                         preferred_element_type=jnp.float32)
         + lax.dot_general(xin, wr_ref[...], dn,
                           preferred_element_type=jnp.float32)
         + b_ref[...])
    r = jnp.where(f_ref[...] > 0.5, jnp.maximum(r, 0.0), r)
    o_ref[0] = r[:, :HD]
    o_ref[1] = r[:, HD:]

  half_spec = pl.BlockSpec((br, HD), lambda i: (i, 0))
  split_spec = pl.BlockSpec((2, br, HD), lambda i: (0, i, 0))
  return pl.pallas_call(
      body,
      grid=(nb,),
      in_specs=[
          half_spec, half_spec,
          pl.BlockSpec((br, 1), lambda i: (i, 0)),
          split_spec,
          pl.BlockSpec((D, D), lambda i: (0, 0)),
          pl.BlockSpec((D, D), lambda i: (0, 0)),
          pl.BlockSpec((1, D), lambda i: (0, 0)),
          pl.BlockSpec((1, 1), lambda i: (0, 0)),
      ],
      out_specs=split_spec,
      out_shape=jax.ShapeDtypeStruct((2, N, HD), jnp.float32),
  )(pa, pb, cnt, x2, w_l, w_r, b_l.reshape(1, D), fl)


def kernel(x, edge_index, W1_l, b1_l, W1_r, W2_l, b2_l, W2_r):
  src = edge_index[0].astype(jnp.int32)
  dst = edge_index[1].astype(jnp.int32)
  packed = src | (dst << IDX_BITS)
  pk = jnp.concatenate(
      [packed, jnp.full((E_PAD - E,), N << IDX_BITS, jnp.int32)]
  ).reshape(NS, K, CHUNK)

  x2 = jnp.stack([x[:, :HD], x[:, HD:]])
  wls = jnp.stack([W1_l, W2_l])
  wrs = jnp.stack([W1_r, W2_r])
  bs = jnp.stack([b1_l, b2_l])
  fls = jnp.array([[[1.0]], [[0.0]]], jnp.float32)

  def step(h2, ws):
    w_l, w_r, b_l, fl = ws
    pa, pb, cnt = _agg(h2, pk)
    h2n = _tc_layer(pa, pb, cnt[:, :1], h2, w_l, w_r, b_l, fl)
    return h2n, 0

  out2, _ = lax.scan(step, x2, (wls, wrs, bs, fls))
  return jnp.concatenate([out2[0], out2[1]], axis=1)
